# baseline (device time: 163652 ns/iter reference)
import jax
import jax.numpy as jnp
from jax import lax
from jax.experimental import pallas as pl
from jax.experimental.pallas import tpu as pltpu

N_DEV = 4
B, SQ, D = 2, 512, 768
HL, DH = 8, 64
HD = HL * DH
NG = 4
GR = 2 * 64


def _prep(x, Wq, K_ext, V_ext, Wo, my):
    Wq_loc = lax.dynamic_slice(Wq, (0, my * HD), (D, HD))
    Wqt = Wq_loc.reshape(D, HL, DH).transpose(1, 0, 2)
    Wo_loc = lax.dynamic_slice(Wo, (my * HD, 0), (HD, D))
    Wot = Wo_loc.reshape(HL, DH, D)

    def group_kv(t):
        return (
            t.reshape(B, 2, NG, 64, HL, DH)
            .transpose(0, 2, 4, 1, 3, 5)
            .reshape(B, NG, HL, GR, DH)
        )

    xg = (
        x.reshape(B, 2, NG, 64, D)
        .transpose(0, 2, 1, 3, 4)
        .reshape(B, NG, GR, D)
    )
    return xg, Wqt, group_kv(K_ext), group_kv(V_ext), Wot


def kernel(x, Wq, K_ext, V_ext, Wo):
    my = lax.axis_index("i")
    xg, Wqt, Kt, Vt, Wot = _prep(x, Wq, K_ext, V_ext, Wo, my)

    def body(xg_ref, wqt_ref, kt_ref, vt_ref, wot_ref, out_ref,
             partial_ref, comm_ref, send_sems, recv_sems):
        me = lax.axis_index("i")
        left = lax.rem(me + N_DEV - 1, N_DEV)
        right = lax.rem(me + 1, N_DEV)

        barrier = pltpu.get_barrier_semaphore()
        for nbr in (left, right):
            pl.semaphore_signal(
                barrier, inc=1,
                device_id=(nbr,), device_id_type=pl.DeviceIdType.MESH,
            )
        pl.semaphore_wait(barrier, 2)

        for b in range(B):
            for g in range(NG):
                xbg = xg_ref[b, g]
                pacc = jnp.zeros((GR, D), jnp.float32)
                for h in range(HL):
                    q = jnp.dot(xbg, wqt_ref[h])
                    k = kt_ref[b, g, h]
                    s = lax.dot_general(
                        q, k, (((1,), (1,)), ((), ()))
                    ) * 0.125
                    s = s - jnp.max(s, axis=-1, keepdims=True)
                    e = jnp.exp(s)
                    w = e / jnp.sum(e, axis=-1, keepdims=True)
                    ctx = jnp.dot(w, vt_ref[b, g, h])
                    pacc = pacc + jnp.dot(ctx, wot_ref[h])
                base = b * SQ
                partial_ref[pl.ds(base + 64 * g, 64), :] = pacc[:64]
                partial_ref[pl.ds(base + 256 + 64 * g, 64), :] = pacc[64:]

        acc = partial_ref[...]
        for h in range(N_DEV - 1):
            src = partial_ref if h == 0 else comm_ref.at[h - 1]
            rdma = pltpu.make_async_remote_copy(
                src_ref=src,
                dst_ref=comm_ref.at[h],
                send_sem=send_sems.at[h],
                recv_sem=recv_sems.at[h],
                device_id=(right,),
                device_id_type=pl.DeviceIdType.MESH,
            )
            rdma.start()
            rdma.wait()
            acc = acc + comm_ref[h]
        out_ref[...] = acc.reshape(B, SQ, D)

    return pl.pallas_call(
        body,
        out_shape=jax.ShapeDtypeStruct((B, SQ, D), jnp.float32),
        in_specs=[pl.BlockSpec(memory_space=pltpu.VMEM)] * 5,
        out_specs=pl.BlockSpec(memory_space=pltpu.VMEM),
        scratch_shapes=[
            pltpu.VMEM((B * SQ, D), jnp.float32),
            pltpu.VMEM((N_DEV - 1, B * SQ, D), jnp.float32),
            pltpu.SemaphoreType.DMA((N_DEV - 1,)),
            pltpu.SemaphoreType.DMA((N_DEV - 1,)),
        ],
        compiler_params=pltpu.CompilerParams(collective_id=0),
    )(xg, Wqt, Kt, Vt, Wot)


# device time: 60262 ns/iter; 2.7157x vs baseline; 2.7157x over previous
import jax
import jax.numpy as jnp
from jax import lax
from jax.experimental import pallas as pl
from jax.experimental.pallas import tpu as pltpu

N_DEV = 4
B, SQ, D = 2, 512, 768
HL, DH = 8, 64
HD = HL * DH
NG = 4
GR = 128
NHOP = 2 * (N_DEV - 1)


def _prep(x, Wq, K_ext, V_ext, Wo, my):
    bf = jnp.bfloat16
    Wq_loc = lax.dynamic_slice(Wq, (0, my * HD), (D, HD)).astype(bf)
    Wo_loc = lax.dynamic_slice(Wo, (my * HD, 0), (HD, D)).astype(bf)

    def group_kv(t):
        return (
            t.reshape(B, 2, NG, 64, HL, DH)
            .transpose(0, 2, 4, 1, 3, 5)
            .reshape(B, NG, HL, GR, DH)
            .astype(bf)
        )

    x2d = (
        x.reshape(B, 2, NG, 64, D)
        .transpose(0, 2, 1, 3, 4)
        .reshape(B * NG * GR, D)
        .astype(bf)
    )
    return x2d, Wq_loc, group_kv(K_ext), group_kv(V_ext), Wo_loc


def kernel(x, Wq, K_ext, V_ext, Wo):
    my = lax.axis_index("i")
    x2d, Wq_loc, Kt, Vt, Wo_loc = _prep(x, Wq, K_ext, V_ext, Wo, my)
    bf = jnp.bfloat16
    f32 = jnp.float32

    def body(x2d_ref, wq_ref, kt_ref, vt_ref, wo_ref, out_ref,
             qfull_ref, partial_ref, cw_recv, ccw_recv, cw_work, ccw_work,
             cw_ssem, cw_rsem, ccw_ssem, ccw_rsem):
        me = lax.axis_index("i")
        left = lax.rem(me + N_DEV - 1, N_DEV)
        right = lax.rem(me + 1, N_DEV)

        barrier = pltpu.get_barrier_semaphore()
        for nbr in (left, right):
            pl.semaphore_signal(
                barrier, inc=1,
                device_id=(nbr,), device_id_type=pl.DeviceIdType.MESH,
            )
        pl.semaphore_wait(barrier, 2)

        qfull_ref[...] = jnp.dot(
            x2d_ref[...], wq_ref[...], preferred_element_type=f32
        ).astype(bf)

        for b in range(B):
            for g in range(NG):
                r0 = (b * NG + g) * GR
                q_all = qfull_ref[r0:r0 + GR, :]
                ctxs = []
                for h in range(HL):
                    q = q_all[:, 64 * h:64 * h + 64]
                    k = kt_ref[b, g, h]
                    s = lax.dot_general(
                        q, k, (((1,), (1,)), ((), ())),
                        preferred_element_type=f32,
                    ) * 0.125
                    s = s - jnp.max(s, axis=-1, keepdims=True)
                    e = jnp.exp(s)
                    w = (e / jnp.sum(e, axis=-1, keepdims=True)).astype(bf)
                    ctxs.append(
                        jnp.dot(w, vt_ref[b, g, h], preferred_element_type=f32)
                    )
                ctx_all = jnp.concatenate(ctxs, axis=1).astype(bf)
                pacc = jnp.dot(ctx_all, wo_ref[...], preferred_element_type=f32)
                partial_ref[b * NG + g] = pacc.astype(bf)

        rdmas = []

        def push(src, dst, ssem, rsem, target):
            r = pltpu.make_async_remote_copy(
                src_ref=src, dst_ref=dst, send_sem=ssem, recv_sem=rsem,
                device_id=(target,), device_id_type=pl.DeviceIdType.MESH,
            )
            r.start()
            rdmas.append(r)
            return r

        def store_out(b, c, val_f32):
            out_ref[b, pl.ds(64 * c, 64), :] = val_f32[:64]
            out_ref[b, pl.ds(256 + 64 * c, 64), :] = val_f32[64:]

        cw = [None] * NHOP
        ccw = [None] * NHOP
        cw[0] = push(partial_ref.at[me], cw_recv.at[0],
                     cw_ssem.at[0], cw_rsem.at[0], right)
        ccw[0] = push(partial_ref.at[N_DEV + me], ccw_recv.at[0],
                      ccw_ssem.at[0], ccw_rsem.at[0], left)

        for h in range(N_DEV - 2):
            cw[h].wait_recv()
            c = lax.rem(me - h - 1 + 2 * N_DEV, N_DEV)
            s = cw_recv[h].astype(f32) + partial_ref[c].astype(f32)
            cw_work[h] = s.astype(bf)
            cw[h + 1] = push(cw_work.at[h], cw_recv.at[h + 1],
                             cw_ssem.at[h + 1], cw_rsem.at[h + 1], right)

            ccw[h].wait_recv()
            c = lax.rem(me + h + 1, N_DEV)
            s = ccw_recv[h].astype(f32) + partial_ref[N_DEV + c].astype(f32)
            ccw_work[h] = s.astype(bf)
            ccw[h + 1] = push(ccw_work.at[h], ccw_recv.at[h + 1],
                              ccw_ssem.at[h + 1], ccw_rsem.at[h + 1], left)

        hf = N_DEV - 2
        cw[hf].wait_recv()
        c_cw = lax.rem(me + 1, N_DEV)
        s = cw_recv[hf].astype(f32) + partial_ref[c_cw].astype(f32)
        store_out(0, c_cw, s)
        cw_work[hf] = s.astype(bf)
        cw[hf + 1] = push(cw_work.at[hf], cw_recv.at[hf + 1],
                          cw_ssem.at[hf + 1], cw_rsem.at[hf + 1], right)

        ccw[hf].wait_recv()
        c_ccw = lax.rem(me + N_DEV - 1, N_DEV)
        s = ccw_recv[hf].astype(f32) + partial_ref[N_DEV + c_ccw].astype(f32)
        store_out(1, c_ccw, s)
        ccw_work[hf] = s.astype(bf)
        ccw[hf + 1] = push(ccw_work.at[hf], ccw_recv.at[hf + 1],
                           ccw_ssem.at[hf + 1], ccw_rsem.at[hf + 1], left)

        for a in range(N_DEV - 1):
            h = N_DEV - 1 + a
            cw[h].wait_recv()
            c = lax.rem(me - a + N_DEV, N_DEV)
            store_out(0, c, cw_recv[h].astype(f32))
            if a < N_DEV - 2:
                cw[h + 1] = push(cw_recv.at[h], cw_recv.at[h + 1],
                                 cw_ssem.at[h + 1], cw_rsem.at[h + 1], right)

            ccw[h].wait_recv()
            c = lax.rem(me + a, N_DEV)
            store_out(1, c, ccw_recv[h].astype(f32))
            if a < N_DEV - 2:
                ccw[h + 1] = push(ccw_recv.at[h], ccw_recv.at[h + 1],
                                  ccw_ssem.at[h + 1], ccw_rsem.at[h + 1], left)

        for r in rdmas:
            r.wait_send()

    return pl.pallas_call(
        body,
        out_shape=jax.ShapeDtypeStruct((B, SQ, D), jnp.float32),
        in_specs=[pl.BlockSpec(memory_space=pltpu.VMEM)] * 5,
        out_specs=pl.BlockSpec(memory_space=pltpu.VMEM),
        scratch_shapes=[
            pltpu.VMEM((B * NG * GR, HD), bf),
            pltpu.VMEM((B * NG, GR, D), bf),
            pltpu.VMEM((NHOP, GR, D), bf),
            pltpu.VMEM((NHOP, GR, D), bf),
            pltpu.VMEM((N_DEV - 1, GR, D), bf),
            pltpu.VMEM((N_DEV - 1, GR, D), bf),
            pltpu.SemaphoreType.DMA((NHOP,)),
            pltpu.SemaphoreType.DMA((NHOP,)),
            pltpu.SemaphoreType.DMA((NHOP,)),
            pltpu.SemaphoreType.DMA((NHOP,)),
        ],
        compiler_params=pltpu.CompilerParams(collective_id=0),
    )(x2d, Wq_loc, Kt, Vt, Wo_loc)


# device time: 19723 ns/iter; 8.2975x vs baseline; 3.0554x over previous
import jax
import jax.numpy as jnp
from jax import lax
from jax.experimental import pallas as pl
from jax.experimental.pallas import tpu as pltpu

N_DEV = 4
B, SQ, D = 2, 512, 768
HL, DH = 8, 64
HD = HL * DH
NG = 4
GR = 128
NHOP = 2 * (N_DEV - 1)


def _prep(x, Wq, K_ext, V_ext, Wo, my):
    bf = jnp.bfloat16
    Wq_loc = (lax.dynamic_slice(Wq, (0, my * HD), (D, HD)) * 0.125).astype(bf)
    Wo_loc = lax.dynamic_slice(Wo, (my * HD, 0), (HD, D)).astype(bf)

    def group_kv(t):
        return (
            t.reshape(B, 2, NG, 64, HL, DH)
            .transpose(0, 2, 4, 1, 3, 5)
            .reshape(B, NG, HL, GR, DH)
            .astype(bf)
        )

    x2d = (
        x.reshape(B, 2, NG, 64, D)
        .transpose(0, 2, 1, 3, 4)
        .reshape(B * NG * GR, D)
        .astype(bf)
    )
    return x2d, Wq_loc, group_kv(K_ext), group_kv(V_ext), Wo_loc


def kernel(x, Wq, K_ext, V_ext, Wo):
    my = lax.axis_index("i")
    x2d, Wq_loc, Kt, Vt, Wo_loc = _prep(x, Wq, K_ext, V_ext, Wo, my)
    bf = jnp.bfloat16
    f32 = jnp.float32

    def body(x2d_ref, wq_ref, kt_ref, vt_ref, wo_ref, out_ref,
             qfull_ref, partial_ref, cw_recv, ccw_recv, cw_work, ccw_work,
             cw_ssem, cw_rsem, ccw_ssem, ccw_rsem):
        me = lax.axis_index("i")
        left = lax.rem(me + N_DEV - 1, N_DEV)
        right = lax.rem(me + 1, N_DEV)

        barrier = pltpu.get_barrier_semaphore()
        for nbr in (left, right):
            pl.semaphore_signal(
                barrier, inc=1,
                device_id=(nbr,), device_id_type=pl.DeviceIdType.MESH,
            )
        pl.semaphore_wait(barrier, 2)

        qfull_ref[...] = jnp.dot(
            x2d_ref[...], wq_ref[...], preferred_element_type=f32
        ).astype(bf)

        def compute_group(b, c):
            r0 = b * NG * GR + c * GR
            q_all = qfull_ref[pl.ds(r0, GR), :]
            ctxs = []
            for h in range(HL):
                q = q_all[:, 64 * h:64 * h + 64]
                k = kt_ref[b, c, h]
                s = lax.dot_general(
                    q, k, (((1,), (1,)), ((), ())),
                    preferred_element_type=f32,
                )
                e = jnp.exp(s)
                w = (e / jnp.sum(e, axis=-1, keepdims=True)).astype(bf)
                ctxs.append(
                    jnp.dot(w, vt_ref[b, c, h], preferred_element_type=f32)
                )
            ctx_all = jnp.concatenate(ctxs, axis=1).astype(bf)
            pacc = jnp.dot(ctx_all, wo_ref[...], preferred_element_type=f32)
            partial_ref[b * NG + c] = pacc.astype(bf)

        rdmas = []

        def push(src, dst, ssem, rsem, target):
            r = pltpu.make_async_remote_copy(
                src_ref=src, dst_ref=dst, send_sem=ssem, recv_sem=rsem,
                device_id=(target,), device_id_type=pl.DeviceIdType.MESH,
            )
            r.start()
            rdmas.append(r)
            return r

        def store_out(b, c, val_f32):
            out_ref[b, pl.ds(64 * c, 64), :] = val_f32[:64]
            out_ref[b, pl.ds(256 + 64 * c, 64), :] = val_f32[64:]

        cw = [None] * NHOP
        ccw = [None] * NHOP

        compute_group(0, me)
        cw[0] = push(partial_ref.at[me], cw_recv.at[0],
                     cw_ssem.at[0], cw_rsem.at[0], right)
        compute_group(1, me)
        ccw[0] = push(partial_ref.at[N_DEV + me], ccw_recv.at[0],
                      ccw_ssem.at[0], ccw_rsem.at[0], left)

        for h in range(N_DEV - 2):
            c_cw = lax.rem(me - h - 1 + 2 * N_DEV, N_DEV)
            c_ccw = lax.rem(me + h + 1, N_DEV)
            compute_group(0, c_cw)
            compute_group(1, c_ccw)

            cw[h].wait_recv()
            s = cw_recv[h].astype(f32) + partial_ref[c_cw].astype(f32)
            cw_work[h] = s.astype(bf)
            cw[h + 1] = push(cw_work.at[h], cw_recv.at[h + 1],
                             cw_ssem.at[h + 1], cw_rsem.at[h + 1], right)

            ccw[h].wait_recv()
            s = ccw_recv[h].astype(f32) + partial_ref[N_DEV + c_ccw].astype(f32)
            ccw_work[h] = s.astype(bf)
            ccw[h + 1] = push(ccw_work.at[h], ccw_recv.at[h + 1],
                              ccw_ssem.at[h + 1], ccw_rsem.at[h + 1], left)

        hf = N_DEV - 2
        c_cw = lax.rem(me + 1, N_DEV)
        c_ccw = lax.rem(me + N_DEV - 1, N_DEV)
        compute_group(0, c_cw)
        compute_group(1, c_ccw)

        cw[hf].wait_recv()
        s_cw = cw_recv[hf].astype(f32) + partial_ref[c_cw].astype(f32)
        cw_work[hf] = s_cw.astype(bf)
        cw[hf + 1] = push(cw_work.at[hf], cw_recv.at[hf + 1],
                          cw_ssem.at[hf + 1], cw_rsem.at[hf + 1], right)
        store_out(0, c_cw, s_cw)

        ccw[hf].wait_recv()
        s_ccw = ccw_recv[hf].astype(f32) + partial_ref[N_DEV + c_ccw].astype(f32)
        ccw_work[hf] = s_ccw.astype(bf)
        ccw[hf + 1] = push(ccw_work.at[hf], ccw_recv.at[hf + 1],
                           ccw_ssem.at[hf + 1], ccw_rsem.at[hf + 1], left)
        store_out(1, c_ccw, s_ccw)

        for a in range(N_DEV - 1):
            h = N_DEV - 1 + a
            cw[h].wait_recv()
            if a < N_DEV - 2:
                cw[h + 1] = push(cw_recv.at[h], cw_recv.at[h + 1],
                                 cw_ssem.at[h + 1], cw_rsem.at[h + 1], right)
            store_out(0, lax.rem(me - a + N_DEV, N_DEV),
                      cw_recv[h].astype(f32))

            ccw[h].wait_recv()
            if a < N_DEV - 2:
                ccw[h + 1] = push(ccw_recv.at[h], ccw_recv.at[h + 1],
                                  ccw_ssem.at[h + 1], ccw_rsem.at[h + 1], left)
            store_out(1, lax.rem(me + a, N_DEV), ccw_recv[h].astype(f32))

        for r in rdmas:
            r.wait_send()

    return pl.pallas_call(
        body,
        out_shape=jax.ShapeDtypeStruct((B, SQ, D), jnp.float32),
        in_specs=[pl.BlockSpec(memory_space=pltpu.VMEM)] * 5,
        out_specs=pl.BlockSpec(memory_space=pltpu.VMEM),
        scratch_shapes=[
            pltpu.VMEM((B * NG * GR, HD), bf),
            pltpu.VMEM((B * NG, GR, D), bf),
            pltpu.VMEM((NHOP, GR, D), bf),
            pltpu.VMEM((NHOP, GR, D), bf),
            pltpu.VMEM((N_DEV - 1, GR, D), bf),
            pltpu.VMEM((N_DEV - 1, GR, D), bf),
            pltpu.SemaphoreType.DMA((NHOP,)),
            pltpu.SemaphoreType.DMA((NHOP,)),
            pltpu.SemaphoreType.DMA((NHOP,)),
            pltpu.SemaphoreType.DMA((NHOP,)),
        ],
        compiler_params=pltpu.CompilerParams(collective_id=0),
    )(x2d, Wq_loc, Kt, Vt, Wo_loc)
